# baseline (speedup 1.0000x reference)
import functools
import math

import jax
import jax.numpy as jnp
from jax import lax
from jax.experimental import pallas as pl
from jax.experimental.pallas import tpu as pltpu
from jax.experimental.pallas import tpu_sc as plsc

D = 64
SCALE = math.sqrt(D)
V = 1000000

_info = plsc.get_sparse_core_info()
NC, NS, L = _info.num_cores, _info.num_subcores, _info.num_lanes
NW = NC * NS

BW = 128
NBLK = V // BW
TAIL0 = NBLK * BW
PNB = 2


def _iota16():
    return lax.iota(jnp.int32, L)


def _splat(v):
    return jnp.full((L,), v, jnp.int32)


def _transpose_scaled(src2d, dst1d, rows, cols, scale, b=None):
    iot = _iota16()
    lead = [] if b is None else [_splat(b)]

    @plsc.parallel_loop(0, L, unroll=4)
    def _(d):
        dm = (iot + d) & (L - 1)
        sd = dm * rows + iot
        for kb in range(rows // L):
            k0 = kb * L
            ridx = iot + k0
            for cb in range(cols // L):
                c0 = cb * L
                vals = plsc.load_gather(src2d, lead + [ridx, dm + c0])
                if scale is not None:
                    vals = vals * scale
                plsc.store_scatter(dst1d, lead + [sd + (c0 * rows + k0)], vals)


@jax.jit
def _table_prep(lut_t, tail_t):
    mesh = plsc.VectorSubcoreMesh(core_axis_name="c", subcore_axis_name="s")

    @functools.partial(
        pl.kernel,
        mesh=mesh,
        compiler_params=pltpu.CompilerParams(needs_layout_passes=False),
        out_type=jax.ShapeDtypeStruct((V * D,), jnp.float32),
        scratch_types=[
            pltpu.VMEM((PNB, D, BW), jnp.float32),
            pltpu.VMEM((PNB, D * BW), jnp.float32),
            pltpu.VMEM((D, D), jnp.float32),
            pltpu.VMEM((D * D,), jnp.float32),
        ]
        + [pltpu.SemaphoreType.DMA] * (2 * PNB),
    )
    def body(src, tail, dst, vbuf, obuf, tin, tout, *sems):
        isems, osems = sems[:PNB], sems[PNB:]
        wid = lax.axis_index("s") * NC + lax.axis_index("c")

        def start_in(t, b):
            c0 = (wid + NW * t) * BW
            pltpu.make_async_copy(
                src.at[:, pl.ds(c0, BW)], vbuf.at[b], isems[b]
            ).start()

        def wait_in(t, b):
            c0 = (wid + NW * t) * BW
            pltpu.make_async_copy(
                src.at[:, pl.ds(c0, BW)], vbuf.at[b], isems[b]
            ).wait()

        def start_out(t, b):
            e0 = (wid + NW * t) * (BW * D)
            pltpu.make_async_copy(
                obuf.at[b], dst.at[pl.ds(e0, BW * D)], osems[b]
            ).start()

        def wait_out(t, b):
            e0 = (wid + NW * t) * (BW * D)
            pltpu.make_async_copy(
                obuf.at[b], dst.at[pl.ds(e0, BW * D)], osems[b]
            ).wait()

        for b in range(PNB):
            @pl.when(wid + NW * b < NBLK)
            def _():
                start_in(b, b)

        def group(tt, _):
            for b in range(PNB):
                t = PNB * tt + b
                blk = wid + NW * t

                @pl.when(blk < NBLK)
                def _():
                    wait_in(t, b)

                    @pl.when(t >= PNB)
                    def _():
                        wait_out(t - PNB, b)

                    _transpose_scaled(vbuf, obuf, D, BW, SCALE, b=b)
                    start_out(t, b)

                    @pl.when(wid + NW * (t + PNB) < NBLK)
                    def _():
                        start_in(t + PNB, b)

            return 0

        lax.fori_loop(0, 123, group, 0)

        nblk_w = (NBLK - wid + NW - 1) // NW
        for b in range(PNB):
            @pl.when(nblk_w > b)
            def _():
                t_b = ((nblk_w - 1 - b) // PNB) * PNB + b
                wait_out(t_b, b)

        @pl.when(wid == 0)
        def _():
            pltpu.sync_copy(tail, tin)
            _transpose_scaled(tin, tout, D, D, SCALE)
            pltpu.sync_copy(tout, dst.at[pl.ds(TAIL0 * D, D * D)])

    return body(lut_t, tail_t)


NB = 4
TPW = 200


@jax.jit
def _gather_tr(tbl, xv):
    mesh = plsc.VectorSubcoreMesh(core_axis_name="c", subcore_axis_name="s")

    @functools.partial(
        pl.kernel,
        mesh=mesh,
        compiler_params=pltpu.CompilerParams(
            use_tc_tiling_on_sc=False, needs_layout_passes=False
        ),
        out_type=jax.ShapeDtypeStruct((200 * 8 * 32 * 8 * 128,), jnp.float32),
        scratch_types=[
            pltpu.VMEM((25, 8, 128), jnp.int32),
            pltpu.VMEM((NB, 128, D), jnp.float32),
            pltpu.VMEM((NB, 128 * D), jnp.float32),
        ]
        + [pltpu.SemaphoreType.DMA] * (2 * NB),
    )
    def body(tbl_hbm, xv_hbm, out_hbm, idx_v, rows_v, tbuf, *sems):
        gsems, wsems = sems[:NB], sems[NB:]
        wid = lax.axis_index("s") * NC + lax.axis_index("c")

        pltpu.sync_copy(xv_hbm.at[pl.ds(wid * 25, 25)], idx_v)

        def task_coords(t):
            lg, jl = t // 8, t % 8
            g = wid * 25 + lg
            jh, ih = g // 32, g % 32
            return lg, jl, jh * 8 + jl, ih

        def start_gather(t, b):
            lg, jl, _, _ = task_coords(t)
            pltpu.make_async_copy(
                tbl_hbm.at[idx_v.at[lg, jl]], rows_v.at[b], gsems[b]
            ).start()

        def wait_gather(t, b):
            lg, jl, _, _ = task_coords(t)
            pltpu.make_async_copy(
                tbl_hbm.at[idx_v.at[lg, jl]], rows_v.at[b], gsems[b]
            ).wait()

        def start_write(t, b):
            _, _, j, ih = task_coords(t)
            for kh in range(8):
                off = ((j * 8 + kh) * 32 + ih) * 1024
                pltpu.make_async_copy(
                    tbuf.at[b, pl.ds(kh * 1024, 1024)],
                    out_hbm.at[pl.ds(off, 1024)],
                    wsems[b],
                ).start()

        def wait_write(t, b):
            _, _, j, ih = task_coords(t)
            for kh in range(8):
                off = ((j * 8 + kh) * 32 + ih) * 1024
                pltpu.make_async_copy(
                    tbuf.at[b, pl.ds(kh * 1024, 1024)],
                    out_hbm.at[pl.ds(off, 1024)],
                    wsems[b],
                ).wait()

        for b in range(NB):
            start_gather(b, b)

        def group(tt, _):
            for b in range(NB):
                t = NB * tt + b

                @pl.when(t < TPW)
                def _():
                    wait_gather(t, b)

                    @pl.when(t >= NB)
                    def _():
                        wait_write(t - NB, b)

                    _transpose_scaled(rows_v, tbuf, 128, D, None, b=b)
                    start_write(t, b)

                    @pl.when(t + NB < TPW)
                    def _():
                        start_gather(t + NB, b)

            return 0

        lax.fori_loop(0, (TPW + NB - 1) // NB, group, 0)

        for b in range(NB):
            wait_write(TPW - NB + b, b)

    return body(tbl, xv)


def kernel(x, lut):
    xi = x.astype(jnp.int32)
    tbl_flat = _table_prep(lut.T, lut[TAIL0:].T)
    tbl = tbl_flat.reshape(V, D)
    xv = xi.reshape(32, 128, 25, 8).transpose(2, 0, 3, 1).reshape(800, 8, 128)
    out5 = _gather_tr(tbl, xv).reshape(200, 8, 32, 8, 128)
    return out5.transpose(2, 4, 0, 1, 3).reshape(4096, 200, D)


# speedup vs baseline: 1.0578x; 1.0578x over previous
import functools
import math

import jax
import jax.numpy as jnp
from jax import lax
from jax.experimental import pallas as pl
from jax.experimental.pallas import tpu as pltpu
from jax.experimental.pallas import tpu_sc as plsc

D = 64
SCALE = math.sqrt(D)
V = 1000000

_info = plsc.get_sparse_core_info()
NC, NS, L = _info.num_cores, _info.num_subcores, _info.num_lanes
NW = NC * NS

BW = 128
NBLK = V // BW
TAIL0 = NBLK * BW
PNB = 2


def _iota16():
    return lax.iota(jnp.int32, L)


def _splat(v):
    return jnp.full((L,), v, jnp.int32)


def _transpose_scaled(src2d, dst1d, rows, cols, scale, b=None):
    iot = _iota16()
    lead = [] if b is None else [_splat(b)]

    @plsc.parallel_loop(0, L, unroll=2)
    def _(d):
        dm = (iot + d) & (L - 1)
        sd = dm * rows + iot
        for kb in range(rows // L):
            k0 = kb * L
            ridx = iot + k0
            for cb in range(cols // L):
                c0 = cb * L
                vals = plsc.load_gather(src2d, lead + [ridx, dm + c0])
                if scale is not None:
                    vals = vals * scale
                plsc.store_scatter(dst1d, lead + [sd + (c0 * rows + k0)], vals)


@jax.jit
def _table_prep(lut_t, tail_t):
    mesh = plsc.VectorSubcoreMesh(core_axis_name="c", subcore_axis_name="s")

    @functools.partial(
        pl.kernel,
        mesh=mesh,
        compiler_params=pltpu.CompilerParams(needs_layout_passes=False),
        out_type=jax.ShapeDtypeStruct((V * D,), jnp.float32),
        scratch_types=[
            pltpu.VMEM((PNB, D, BW), jnp.float32),
            pltpu.VMEM((PNB, D * BW), jnp.float32),
            pltpu.VMEM((D, D), jnp.float32),
            pltpu.VMEM((D * D,), jnp.float32),
        ]
        + [pltpu.SemaphoreType.DMA] * (2 * PNB),
    )
    def body(src, tail, dst, vbuf, obuf, tin, tout, *sems):
        isems, osems = sems[:PNB], sems[PNB:]
        wid = lax.axis_index("s") * NC + lax.axis_index("c")

        def start_in(t, b):
            c0 = (wid + NW * t) * BW
            pltpu.make_async_copy(
                src.at[:, pl.ds(c0, BW)], vbuf.at[b], isems[b]
            ).start()

        def wait_in(t, b):
            c0 = (wid + NW * t) * BW
            pltpu.make_async_copy(
                src.at[:, pl.ds(c0, BW)], vbuf.at[b], isems[b]
            ).wait()

        def start_out(t, b):
            e0 = (wid + NW * t) * (BW * D)
            pltpu.make_async_copy(
                obuf.at[b], dst.at[pl.ds(e0, BW * D)], osems[b]
            ).start()

        def wait_out(t, b):
            e0 = (wid + NW * t) * (BW * D)
            pltpu.make_async_copy(
                obuf.at[b], dst.at[pl.ds(e0, BW * D)], osems[b]
            ).wait()

        for b in range(PNB):
            @pl.when(wid + NW * b < NBLK)
            def _():
                start_in(b, b)

        def group(tt, _):
            for b in range(PNB):
                t = PNB * tt + b
                blk = wid + NW * t

                @pl.when(blk < NBLK)
                def _():
                    wait_in(t, b)

                    @pl.when(t >= PNB)
                    def _():
                        wait_out(t - PNB, b)

                    _transpose_scaled(vbuf, obuf, D, BW, SCALE, b=b)
                    start_out(t, b)

                    @pl.when(wid + NW * (t + PNB) < NBLK)
                    def _():
                        start_in(t + PNB, b)

            return 0

        lax.fori_loop(0, 123, group, 0)

        nblk_w = (NBLK - wid + NW - 1) // NW
        for b in range(PNB):
            @pl.when(nblk_w > b)
            def _():
                t_b = ((nblk_w - 1 - b) // PNB) * PNB + b
                wait_out(t_b, b)

        @pl.when(wid == 0)
        def _():
            pltpu.sync_copy(tail, tin)
            _transpose_scaled(tin, tout, D, D, SCALE)
            pltpu.sync_copy(tout, dst.at[pl.ds(TAIL0 * D, D * D)])

    return body(lut_t, tail_t)


NB = 4
TPW = 200


@jax.jit
def _gather_tr(tbl, xv):
    mesh = plsc.VectorSubcoreMesh(core_axis_name="c", subcore_axis_name="s")

    @functools.partial(
        pl.kernel,
        mesh=mesh,
        compiler_params=pltpu.CompilerParams(
            use_tc_tiling_on_sc=False, needs_layout_passes=False
        ),
        out_type=jax.ShapeDtypeStruct((200 * 8 * 32 * 8 * 128,), jnp.float32),
        scratch_types=[
            pltpu.VMEM((25, 8, 128), jnp.int32),
            pltpu.VMEM((NB, 128, D), jnp.float32),
            pltpu.VMEM((NB, 128 * D), jnp.float32),
        ]
        + [pltpu.SemaphoreType.DMA] * (2 * NB),
    )
    def body(tbl_hbm, xv_hbm, out_hbm, idx_v, rows_v, tbuf, *sems):
        gsems, wsems = sems[:NB], sems[NB:]
        wid = lax.axis_index("s") * NC + lax.axis_index("c")

        pltpu.sync_copy(xv_hbm.at[pl.ds(wid * 25, 25)], idx_v)

        def task_coords(t):
            lg, jl = t // 8, t % 8
            g = wid * 25 + lg
            jh, ih = g // 32, g % 32
            return lg, jl, jh * 8 + jl, ih

        def start_gather(t, b):
            lg, jl, _, _ = task_coords(t)
            pltpu.make_async_copy(
                tbl_hbm.at[idx_v.at[lg, jl]], rows_v.at[b], gsems[b]
            ).start()

        def wait_gather(t, b):
            lg, jl, _, _ = task_coords(t)
            pltpu.make_async_copy(
                tbl_hbm.at[idx_v.at[lg, jl]], rows_v.at[b], gsems[b]
            ).wait()

        def start_write(t, b):
            _, _, j, ih = task_coords(t)
            for kh in range(8):
                off = ((j * 8 + kh) * 32 + ih) * 1024
                pltpu.make_async_copy(
                    tbuf.at[b, pl.ds(kh * 1024, 1024)],
                    out_hbm.at[pl.ds(off, 1024)],
                    wsems[b],
                ).start()

        def wait_write(t, b):
            _, _, j, ih = task_coords(t)
            for kh in range(8):
                off = ((j * 8 + kh) * 32 + ih) * 1024
                pltpu.make_async_copy(
                    tbuf.at[b, pl.ds(kh * 1024, 1024)],
                    out_hbm.at[pl.ds(off, 1024)],
                    wsems[b],
                ).wait()

        for b in range(NB):
            start_gather(b, b)

        def group(tt, _):
            for b in range(NB):
                t = NB * tt + b

                @pl.when(t < TPW)
                def _():
                    wait_gather(t, b)

                    @pl.when(t >= NB)
                    def _():
                        wait_write(t - NB, b)

                    _transpose_scaled(rows_v, tbuf, 128, D, None, b=b)
                    start_write(t, b)

                    @pl.when(t + NB < TPW)
                    def _():
                        start_gather(t + NB, b)

            return 0

        lax.fori_loop(0, (TPW + NB - 1) // NB, group, 0)

        for b in range(NB):
            wait_write(TPW - NB + b, b)

    return body(tbl, xv)


def kernel(x, lut):
    xi = x.astype(jnp.int32)
    tbl_flat = _table_prep(lut.T, lut[TAIL0:].T)
    tbl = tbl_flat.reshape(V, D)
    xv = xi.reshape(32, 128, 25, 8).transpose(2, 0, 3, 1).reshape(800, 8, 128)
    out5 = _gather_tr(tbl, xv).reshape(200, 8, 32, 8, 128)
    return out5.transpose(2, 4, 0, 1, 3).reshape(4096, 200, D)
